# baseline (speedup 1.0000x reference)
import functools

import jax
import jax.numpy as jnp
from jax import lax
from jax.experimental import pallas as pl
from jax.experimental.pallas import tpu as pltpu
from jax.experimental.pallas import tpu_sc as plsc

NC = 2
NS = 16
NW = NC * NS
EMBED = 128
CB = 4
NBUF = 3


def kernel(x, table):
    batch, hist = x.shape
    idx = x.reshape(batch * hist).astype(jnp.int32)
    rows_per_worker = batch // NW
    n_chunks = rows_per_worker // CB
    chunk_idx = CB * hist
    worker_idx = rows_per_worker * hist
    n_loop = (n_chunks // 6) * 6

    mesh = plsc.VectorSubcoreMesh(core_axis_name="c", subcore_axis_name="s")

    @functools.partial(
        pl.kernel,
        mesh=mesh,
        out_type=jax.ShapeDtypeStruct((batch, hist, EMBED), table.dtype),
        scratch_types=[
            pltpu.VMEM((chunk_idx,), jnp.int32),
            pltpu.VMEM((chunk_idx,), jnp.int32),
            pltpu.VMEM((chunk_idx,), jnp.int32),
            pltpu.VMEM((chunk_idx, EMBED), table.dtype),
            pltpu.VMEM((chunk_idx, EMBED), table.dtype),
            pltpu.VMEM((chunk_idx, EMBED), table.dtype),
            pltpu.VMEM_SHARED((NS, chunk_idx, EMBED), table.dtype),
            pltpu.VMEM_SHARED((NS, chunk_idx, EMBED), table.dtype),
            pltpu.SemaphoreType.DMA,
            pltpu.SemaphoreType.DMA,
            pltpu.SemaphoreType.DMA,
            pltpu.SemaphoreType.DMA,
            pltpu.SemaphoreType.DMA,
            pltpu.SemaphoreType.DMA,
            pltpu.SemaphoreType.DMA,
            pltpu.SemaphoreType.DMA,
        ],
    )
    def embed_kernel(
        tab_hbm, idx_hbm, out_hbm, i0, i1, i2,
        r0, r1, r2, s0, s1,
        g0, g1, g2, c0_, c1_, c2_, o0, o1,
    ):
        idx_v = (i0, i1, i2)
        rows_v = (r0, r1, r2)
        spm = (s0, s1)
        gsem = (g0, g1, g2)
        csem = (c0_, c1_, c2_)
        osem = (o0, o1)
        sid = lax.axis_index("s")
        wid = lax.axis_index("c") * NS + sid
        base_row = wid * rows_per_worker

        def issue(c, b):
            off = (base_row + c * CB) * hist
            pltpu.sync_copy(idx_hbm.at[pl.ds(off, chunk_idx)], idx_v[b])
            pltpu.async_copy(tab_hbm.at[idx_v[b]], rows_v[b], gsem[b])

        def wait_gather(c, b):
            pltpu.make_async_copy(
                tab_hbm.at[idx_v[b]], rows_v[b], gsem[b]
            ).wait()

        def stage(b, s):
            pltpu.async_copy(rows_v[b], spm[s].at[sid], csem[b])

        def wait_stage(b, s):
            pltpu.make_async_copy(rows_v[b], spm[s].at[sid], csem[b]).wait()

        def fire_out(c, s):
            for j in range(CB):
                row = base_row + c * CB + j
                pltpu.async_copy(
                    spm[s].at[sid, pl.ds(j * hist, hist)],
                    out_hbm.at[row],
                    osem[s],
                )

        def drain_out(s):
            for j in range(CB):
                pltpu.make_async_copy(
                    spm[s].at[sid, pl.ds(j * hist, hist)],
                    out_hbm.at[base_row],
                    osem[s],
                ).wait()

        def chunk_step(c, b, s):
            bn = (b + 2) % NBUF
            sn = (s + 1) % 2
            wait_gather(c, b)

            @pl.when(c >= 2)
            def _():
                drain_out(s)

            stage(b, s)

            @pl.when(c >= 1)
            def _():
                wait_stage(bn, sn)
                fire_out(c - 1, sn)

            @pl.when(c + 2 < n_chunks)
            def _():
                issue(c + 2, bn)

        issue(0, 0)
        issue(1, 1)

        @pl.loop(0, n_loop, step=6)
        def _(c0):
            for k in range(6):
                chunk_step(c0 + k, k % NBUF, k % 2)

        for c_tail in range(n_loop, n_chunks):
            chunk_step(jnp.int32(c_tail), c_tail % NBUF, c_tail % 2)

        last = n_chunks - 1
        wait_stage(last % NBUF, last % 2)
        fire_out(jnp.int32(last), last % 2)
        drain_out((last + 1) % 2)
        drain_out(last % 2)

    return embed_kernel(table, idx)
